# baseline (speedup 1.0000x reference)
import jax
import jax.numpy as jnp
from jax import lax
from jax.experimental import pallas as pl
from jax.experimental.pallas import tpu as pltpu
from jax.experimental.pallas import tpu_sc as plsc

EMB_DIM = 64
BATCH = 16384

_info = plsc.get_sparse_core_info()
_NC, _NS, _L = _info.num_cores, _info.num_subcores, _info.num_lanes
_NW = _NC * _NS
_BPW = BATCH // _NW


def _sc_body(sbj_hbm, idx_hbm, wr_hbm, out_hbm, idx_v, rows_v, sbj_v, out_v,
             pscr_v, sem_g, sem_s):
    wid = lax.axis_index("s") * _NC + lax.axis_index("c")
    base = wid * _BPW
    pltpu.sync_copy(idx_hbm.at[pl.ds(base, _BPW)], idx_v)
    gat = pltpu.async_copy(wr_hbm.at[idx_v], rows_v, sem_g)
    cps = pltpu.async_copy(sbj_hbm.at[pl.ds(base, _BPW)], sbj_v, sem_s)
    gat.wait()
    cps.wait()

    lane = lax.iota(jnp.int32, _L)

    def group(g, carry):
        for jj in range(_L):
            j = g * _L + jj
            acc = jnp.zeros((_L,), jnp.float32)
            for c in range(EMB_DIM // _L):
                s = sbj_v[j, pl.ds(c * _L, _L)]
                r = rows_v[j, pl.ds(c * _L, _L)]
                acc = acc + s * (r * r)
            pscr_v[pl.ds(jj * _L, _L)] = acc
        tot = jnp.zeros((_L,), jnp.float32)
        for d in range(_L):
            col = plsc.load_gather(pscr_v, [lane * _L + d])
            tot = tot + col
        out_v[pl.ds(g * _L, _L)] = tot
        return carry

    lax.fori_loop(0, _BPW // _L, group, 0)
    pltpu.sync_copy(out_v, out_hbm.at[pl.ds(base, _BPW)])


def kernel(sbj_embs, obj_embs, rel_ids, W_r):
    mesh = plsc.VectorSubcoreMesh(core_axis_name="c", subcore_axis_name="s")
    k = pl.kernel(
        _sc_body,
        mesh=mesh,
        compiler_params=pltpu.CompilerParams(
            needs_layout_passes=False, use_tc_tiling_on_sc=False),
        out_type=jax.ShapeDtypeStruct((BATCH,), jnp.float32),
        scratch_types=[
            pltpu.VMEM((_BPW,), jnp.int32),
            pltpu.VMEM((_BPW, EMB_DIM), jnp.float32),
            pltpu.VMEM((_BPW, EMB_DIM), jnp.float32),
            pltpu.VMEM((_BPW,), jnp.float32),
            pltpu.VMEM((_L * _L,), jnp.float32),
            pltpu.SemaphoreType.DMA,
            pltpu.SemaphoreType.DMA,
        ],
    )
    return k(sbj_embs, rel_ids.astype(jnp.int32), W_r)
